# baseline (speedup 1.0000x reference)
import functools

import jax
import jax.numpy as jnp
from jax import lax
from jax.experimental import pallas as pl
from jax.experimental.pallas import tpu as pltpu
from jax.experimental.pallas import tpu_sc as plsc

NC = 2
NS = 16
L = 16
D = 64
N_ROWS = 50000
HALF = N_ROWS // NC
LPT = 1568
LOCAL = NS * LPT

CHUNK = 128
BLK = 20
N_EDGES = 800000
BLK_EDGES = NS * BLK * CHUNK
NBLK_TOT = 21
E_TOT = NBLK_TOT * BLK_EDGES


def _spmm_body(x_hbm, row_hbm, col_hbm, val_hbm, out_hbm,
               acc, ridx_b, cidx_b, val_b,
               lidx0, lidx1, rows0, rows1,
               semg0, semg1, sems0, sems1):
    c = lax.axis_index("c")
    s = lax.axis_index("s")
    lidx = (lidx0, lidx1)
    rows = (rows0, rows1)
    semg = (semg0, semg1)
    sems = (sems0, sems1)

    zv = jnp.zeros((L,), jnp.float32)

    def zfill(j, _):
        for k in range(D // L):
            rows0[j, pl.ds(k * L, L)] = zv
        return 0

    lax.fori_loop(0, 112, zfill, 0)

    def zcopy(j, _):
        pltpu.sync_copy(rows0.at[pl.ds(0, 112)],
                        acc.at[pl.ds(s * LPT + j * 112, 112)])
        return 0

    lax.fori_loop(0, LPT // 112, zcopy, 0)
    plsc.subcore_barrier()

    row_off = c * HALF

    def gather_start(g, b):
        pltpu.async_copy(
            x_hbm.at[cidx_b.at[pl.ds(g * CHUNK, CHUNK)]], rows[b], semg[b]
        )

    def gather_wait(b):
        pltpu.make_async_copy(
            x_hbm.at[cidx_b.at[pl.ds(0, CHUNK)]], rows[b], semg[b]
        ).wait()

    def scatter_start(b):
        pltpu.async_copy(rows[b], acc.at[lidx[b]], sems[b], add=True)

    def scatter_wait(b):
        pltpu.make_async_copy(rows[b], acc.at[lidx[b]], sems[b]).wait()

    def compute(g, b):
        def mloop(t, _):
            r16 = ridx_b[pl.ds(g * CHUNK + t * L, L)] - row_off
            ok = (r16 >= 0) & (r16 < HALF)
            lidx[b][pl.ds(t * L, L)] = jnp.where(ok, r16, HALF)
            return 0

        lax.fori_loop(0, CHUNK // L, mloop, 0)

        def eloop(t, _):
            bv16 = val_b[pl.ds(g * CHUNK + t * L, L)]
            for j in range(L):
                bv = jnp.full((L,), bv16[j], jnp.float32)
                e = t * L + j
                for k in range(D // L):
                    rows[b][e, pl.ds(k * L, L)] = (
                        rows[b][e, pl.ds(k * L, L)] * bv
                    )
            return 0

        lax.fori_loop(0, CHUNK // L, eloop, 0)

    def block_body(blk, _):
        eb = blk * BLK_EDGES + s * (BLK * CHUNK)
        pltpu.sync_copy(row_hbm.at[pl.ds(eb, BLK * CHUNK)], ridx_b)
        pltpu.sync_copy(col_hbm.at[pl.ds(eb, BLK * CHUNK)], cidx_b)
        pltpu.sync_copy(val_hbm.at[pl.ds(eb, BLK * CHUNK)], val_b)

        gather_start(0, 0)

        def pair_body(p, _):
            for b in range(2):
                g = 2 * p + b
                nb = 1 - b

                @pl.when(g + 1 < BLK)
                def _():
                    @pl.when(g >= 1)
                    def _():
                        scatter_wait(nb)

                    gather_start(g + 1, nb)

                gather_wait(b)
                compute(g, b)
                scatter_start(b)
            return 0

        lax.fori_loop(0, BLK // 2, pair_body, 0)
        scatter_wait(0)
        scatter_wait(1)
        return 0

    lax.fori_loop(0, NBLK_TOT, block_body, 0)
    plsc.subcore_barrier()

    pltpu.sync_copy(acc.at[pl.ds(s * LPT, LPT)], out_hbm.at[c, pl.ds(s * LPT, LPT)])


def _spmm(x, row, col, val):
    f = pl.kernel(
        _spmm_body,
        out_type=jax.ShapeDtypeStruct((NC, LOCAL, D), jnp.float32),
        mesh=plsc.VectorSubcoreMesh(core_axis_name="c", subcore_axis_name="s"),
        scratch_types=[
            pltpu.VMEM_SHARED((LOCAL, D), jnp.float32),
            pltpu.VMEM((BLK * CHUNK,), jnp.int32),
            pltpu.VMEM((BLK * CHUNK,), jnp.int32),
            pltpu.VMEM((BLK * CHUNK,), jnp.float32),
            pltpu.VMEM((CHUNK,), jnp.int32),
            pltpu.VMEM((CHUNK,), jnp.int32),
            pltpu.VMEM((CHUNK, D), jnp.float32),
            pltpu.VMEM((CHUNK, D), jnp.float32),
            pltpu.SemaphoreType.DMA,
            pltpu.SemaphoreType.DMA,
            pltpu.SemaphoreType.DMA,
            pltpu.SemaphoreType.DMA,
        ],
        compiler_params=pltpu.CompilerParams(use_tc_tiling_on_sc=False),
    )
    o = f(x, row, col, val)
    return o[:, :HALF, :].reshape(N_ROWS, D)


def _pad(dest, other, val):
    pad = E_TOT - N_EDGES
    dest_p = jnp.concatenate([dest, jnp.zeros((pad,), jnp.int32)])
    other_p = jnp.concatenate([other, jnp.zeros((pad,), jnp.int32)])
    val_p = jnp.concatenate([val, jnp.zeros((pad,), jnp.float32)])
    return dest_p, other_p, val_p


def _finish_body(u0, u1, u2, u3, i0, i1, i2, i3, s1, s2, fu_ref, fi_ref):
    u = (u0[...] + u1[...] + u2[...] + u3[...]) * 0.25
    sm = (u0[...] + s1[...] + s2[...]) * (1.0 / 3.0)
    it = (i0[...] + i1[...] + i2[...] + i3[...]) * 0.25
    fu = 0.6 * u + 0.4 * sm
    nu = jnp.sqrt(jnp.sum(fu * fu, axis=1, keepdims=True))
    fu_ref[...] = fu / jnp.maximum(nu, 1e-12)
    ni = jnp.sqrt(jnp.sum(it * it, axis=1, keepdims=True))
    fi_ref[...] = it / jnp.maximum(ni, 1e-12)


def _finish(us, its, ss):
    bspec = pl.BlockSpec((1000, D), lambda i: (i, 0))
    return pl.pallas_call(
        _finish_body,
        grid=(N_ROWS // 1000,),
        in_specs=[bspec] * 10,
        out_specs=[bspec, bspec],
        out_shape=[jax.ShapeDtypeStruct((N_ROWS, D), jnp.float32)] * 2,
    )(us[0], us[1], us[2], us[3], its[0], its[1], its[2], its[3], ss[1], ss[2])


def kernel(user_table, item_table, r_edge_index, r_values, rt_values,
           s_edge_index, s_values):
    rr = r_edge_index[0]
    rc = r_edge_index[1]
    sr = s_edge_index[0]
    sc = s_edge_index[1]

    ar, ac, av = _pad(rr, rc, r_values)
    br, bc, bv = _pad(rc, rr, rt_values)
    dr, dc, dv = _pad(sr, sc, s_values)

    cu, ci = user_table, item_table
    us, its = [cu], [ci]
    for _ in range(3):
        nu = _spmm(ci, ar, ac, av)
        ni = _spmm(cu, br, bc, bv)
        cu, ci = nu, ni
        us.append(cu)
        its.append(ci)

    cs = user_table
    ss = [cs]
    for _ in range(2):
        cs = _spmm(cs, dr, dc, dv)
        ss.append(cs)

    fu, fi = _finish(us, its, ss)
    return (fu, fi)


# speedup vs baseline: 2.3830x; 2.3830x over previous
import functools

import jax
import jax.numpy as jnp
from jax import lax
from jax.experimental import pallas as pl
from jax.experimental.pallas import tpu as pltpu
from jax.experimental.pallas import tpu_sc as plsc

NC = 2
NS = 16
L = 16
D = 64
N_ROWS = 50000
HALF = N_ROWS // NC
LPT = 1568
LOCAL = NS * LPT

CHUNK = 128
BLK = 20
N_EDGES = 800000
E_PAD = 819200
EDGES_PER_TILE = E_PAD // NS
BLOCKS_PER_TILE = EDGES_PER_TILE // (BLK * CHUNK)


def _spmm_body(x_hbm, row_hbm, col_hbm, val_hbm, out_hbm,
               acc, ridx_b, cidx_b, val_b,
               lidx0, lidx1, rows0, rows1,
               semg0, semg1, sems0, sems1):
    c = lax.axis_index("c")
    s = lax.axis_index("s")
    lidx = (lidx0, lidx1)
    rows = (rows0, rows1)
    semg = (semg0, semg1)
    sems = (sems0, sems1)

    zv = jnp.zeros((L,), jnp.float32)

    def zfill(j, _):
        for k in range(D // L):
            rows0[j, pl.ds(k * L, L)] = zv
        return 0

    lax.fori_loop(0, 112, zfill, 0)

    def zcopy(j, _):
        pltpu.sync_copy(rows0.at[pl.ds(0, 112)],
                        acc.at[pl.ds(s * LPT + j * 112, 112)])
        return 0

    lax.fori_loop(0, LPT // 112, zcopy, 0)
    plsc.subcore_barrier()

    ebase = s * EDGES_PER_TILE
    row_off = c * HALF

    def gather_start(g, b):
        pltpu.async_copy(
            x_hbm.at[cidx_b.at[pl.ds(g * CHUNK, CHUNK)]], rows[b], semg[b]
        )

    def gather_wait(b):
        pltpu.make_async_copy(
            x_hbm.at[cidx_b.at[pl.ds(0, CHUNK)]], rows[b], semg[b]
        ).wait()

    def scatter_start(b):
        pltpu.async_copy(rows[b], acc.at[lidx[b]], sems[b], add=True)

    def scatter_wait(b):
        pltpu.make_async_copy(rows[b], acc.at[lidx[b]], sems[b]).wait()

    def compute(g, b):
        def mloop(t, _):
            r16 = ridx_b[pl.ds(g * CHUNK + t * L, L)] - row_off
            ok = (r16 >= 0) & (r16 < HALF)
            lidx[b][pl.ds(t * L, L)] = jnp.where(ok, r16, HALF)
            return 0

        lax.fori_loop(0, CHUNK // L, mloop, 0)

        def eloop(t, _):
            bv16 = val_b[pl.ds(g * CHUNK + t * L, L)]
            for j in range(L):
                bv = jnp.full((L,), bv16[j], jnp.float32)
                e = t * L + j
                for k in range(D // L):
                    rows[b][e, pl.ds(k * L, L)] = (
                        rows[b][e, pl.ds(k * L, L)] * bv
                    )
            return 0

        lax.fori_loop(0, CHUNK // L, eloop, 0)

    def block_body(blk, _):
        eb = ebase + blk * (BLK * CHUNK)
        pltpu.sync_copy(row_hbm.at[pl.ds(eb, BLK * CHUNK)], ridx_b)
        pltpu.sync_copy(col_hbm.at[pl.ds(eb, BLK * CHUNK)], cidx_b)
        pltpu.sync_copy(val_hbm.at[pl.ds(eb, BLK * CHUNK)], val_b)

        gather_start(0, 0)

        def pair_body(p, _):
            for b in range(2):
                g = 2 * p + b
                nb = 1 - b

                @pl.when(g + 1 < BLK)
                def _():
                    @pl.when(g >= 1)
                    def _():
                        scatter_wait(nb)

                    gather_start(g + 1, nb)

                gather_wait(b)
                compute(g, b)
                scatter_start(b)
            return 0

        lax.fori_loop(0, BLK // 2, pair_body, 0)
        scatter_wait(0)
        scatter_wait(1)
        return 0

    lax.fori_loop(0, BLOCKS_PER_TILE, block_body, 0)
    plsc.subcore_barrier()

    pltpu.sync_copy(acc.at[pl.ds(s * LPT, LPT)], out_hbm.at[c, pl.ds(s * LPT, LPT)])


def _spmm(x, row, col, val):
    f = pl.kernel(
        _spmm_body,
        out_type=jax.ShapeDtypeStruct((NC, LOCAL, D), jnp.float32),
        mesh=plsc.VectorSubcoreMesh(core_axis_name="c", subcore_axis_name="s"),
        scratch_types=[
            pltpu.VMEM_SHARED((LOCAL, D), jnp.float32),
            pltpu.VMEM((BLK * CHUNK,), jnp.int32),
            pltpu.VMEM((BLK * CHUNK,), jnp.int32),
            pltpu.VMEM((BLK * CHUNK,), jnp.float32),
            pltpu.VMEM((CHUNK,), jnp.int32),
            pltpu.VMEM((CHUNK,), jnp.int32),
            pltpu.VMEM((CHUNK, D), jnp.float32),
            pltpu.VMEM((CHUNK, D), jnp.float32),
            pltpu.SemaphoreType.DMA,
            pltpu.SemaphoreType.DMA,
            pltpu.SemaphoreType.DMA,
            pltpu.SemaphoreType.DMA,
        ],
        compiler_params=pltpu.CompilerParams(use_tc_tiling_on_sc=False),
    )
    o = f(x, row, col, val)
    return o[:, :HALF, :].reshape(N_ROWS, D)


def _finish_body(u0, u1, u2, u3, i0, i1, i2, i3, s1, s2, fu_ref, fi_ref):
    u = (u0[...] + u1[...] + u2[...] + u3[...]) * 0.25
    sm = (u0[...] + s1[...] + s2[...]) * (1.0 / 3.0)
    it = (i0[...] + i1[...] + i2[...] + i3[...]) * 0.25
    fu = 0.6 * u + 0.4 * sm
    nu = jnp.sqrt(jnp.sum(fu * fu, axis=1, keepdims=True))
    fu_ref[...] = fu / jnp.maximum(nu, 1e-12)
    ni = jnp.sqrt(jnp.sum(it * it, axis=1, keepdims=True))
    fi_ref[...] = it / jnp.maximum(ni, 1e-12)


def _finish(us, its, ss):
    bspec = pl.BlockSpec((1000, D), lambda i: (i, 0))
    return pl.pallas_call(
        _finish_body,
        grid=(N_ROWS // 1000,),
        in_specs=[bspec] * 10,
        out_specs=[bspec, bspec],
        out_shape=[jax.ShapeDtypeStruct((N_ROWS, D), jnp.float32)] * 2,
    )(us[0], us[1], us[2], us[3], its[0], its[1], its[2], its[3], ss[1], ss[2])


def kernel(user_table, item_table, r_edge_index, r_values, rt_values,
           s_edge_index, s_values):
    pad = E_PAD - N_EDGES
    rr = jnp.pad(r_edge_index[0], (0, pad))
    rc = jnp.pad(r_edge_index[1], (0, pad))
    rv = jnp.pad(r_values, (0, pad))
    rtv = jnp.pad(rt_values, (0, pad))
    sr = jnp.pad(s_edge_index[0], (0, pad))
    sc = jnp.pad(s_edge_index[1], (0, pad))
    sv = jnp.pad(s_values, (0, pad))

    cu, ci = user_table, item_table
    us, its = [cu], [ci]
    for _ in range(3):
        nu = _spmm(ci, rr, rc, rv)
        ni = _spmm(cu, rc, rr, rtv)
        cu, ci = nu, ni
        us.append(cu)
        its.append(ci)

    cs = user_table
    ss = [cs]
    for _ in range(2):
        cs = _spmm(cs, sr, sc, sv)
        ss.append(cs)

    fu, fi = _finish(us, its, ss)
    return (fu, fi)
